# Initial kernel scaffold; baseline (speedup 1.0000x reference)
import jax
import jax.numpy as jnp
from jax.experimental import pallas as pl


def kernel(x, edge_index, edge_attr, batch, W0, b0, We1, be1, We2, be2, Wroot, bconv, gru_Wih, gru_Whh, gru_bih, gru_bhh, lstm_Wih, lstm_Whh, lstm_bih, lstm_bhh, Wf1, bf1, Wf2, bf2):
    raise NotImplementedError("write your pallas kernel here")


# speedup vs baseline: 2.5373x; 2.5373x over previous
import functools

import jax
import jax.numpy as jnp
from jax import lax
from jax.experimental import pallas as pl
from jax.experimental.pallas import tpu as pltpu
from jax.experimental.pallas import tpu_sc as plsc

_NC = 2
_NS = 16
_NW = _NC * _NS

_CHUNK = 1000

_MSG_TILE = 800


def _sc_mesh():
    return plsc.VectorSubcoreMesh(
        core_axis_name="c", subcore_axis_name="s", num_cores=_NC, num_subcores=_NS
    )


_SC_PARAMS = pltpu.CompilerParams(use_tc_tiling_on_sc=False)


def _sc_deg(dst, ones_c, zeros_n):
    N = zeros_n.shape[0]
    E = dst.shape[0]
    epw = E // _NW
    nch = epw // _CHUNK

    @functools.partial(
        pl.kernel,
        out_type=jax.ShapeDtypeStruct((_NC, N), jnp.float32),
        mesh=_sc_mesh(),
        compiler_params=_SC_PARAMS,
        scratch_types=[
            pltpu.VMEM((_CHUNK,), jnp.int32),
            pltpu.VMEM((_CHUNK,), jnp.float32),
            pltpu.VMEM_SHARED((N,), jnp.float32),
        ],
    )
    def body(dst_hbm, ones_hbm, zeros_hbm, out_hbm, idx_v, ones_v, acc_s):
        c = lax.axis_index("c")
        s = lax.axis_index("s")
        wid = s * _NC + c

        @pl.when(s == 0)
        def _():
            pltpu.sync_copy(zeros_hbm, acc_s)

        pltpu.sync_copy(ones_hbm, ones_v)
        plsc.subcore_barrier()
        for k in range(nch):
            base = wid * epw + k * _CHUNK
            pltpu.sync_copy(dst_hbm.at[pl.ds(base, _CHUNK)], idx_v)
            pltpu.sync_copy(ones_v, acc_s.at[idx_v], add=True)
        plsc.subcore_barrier()

        @pl.when(s == 0)
        def _():
            pltpu.sync_copy(acc_s, out_hbm.at[c])

    return body(dst, ones_c, zeros_n)


def _sc_gather_rows(table, idx):
    N, G = table.shape
    E = idx.shape[0]
    epw = E // _NW
    nch = epw // _CHUNK

    @functools.partial(
        pl.kernel,
        out_type=jax.ShapeDtypeStruct((E, G), jnp.float32),
        mesh=_sc_mesh(),
        compiler_params=_SC_PARAMS,
        scratch_types=[
            pltpu.VMEM((_CHUNK,), jnp.int32),
            pltpu.VMEM((_CHUNK, G), jnp.float32),
            pltpu.SemaphoreType.DMA,
        ],
    )
    def body(table_hbm, idx_hbm, out_hbm, idx_v, rows_v, sem):
        c = lax.axis_index("c")
        s = lax.axis_index("s")
        wid = s * _NC + c
        for k in range(nch):
            base = wid * epw + k * _CHUNK
            pltpu.sync_copy(idx_hbm.at[pl.ds(base, _CHUNK)], idx_v)
            pltpu.async_copy(table_hbm.at[idx_v], rows_v, sem).wait()
            pltpu.sync_copy(rows_v, out_hbm.at[pl.ds(base, _CHUNK)])

    return body(table, idx)


def _sc_gather_scalar(table, idx):
    N = table.shape[0]
    E = idx.shape[0]
    epw = E // _NW
    nch = epw // _CHUNK

    @functools.partial(
        pl.kernel,
        out_type=jax.ShapeDtypeStruct((E,), jnp.float32),
        mesh=_sc_mesh(),
        compiler_params=_SC_PARAMS,
        scratch_types=[
            pltpu.VMEM((_CHUNK,), jnp.int32),
            pltpu.VMEM((_CHUNK,), jnp.float32),
            pltpu.SemaphoreType.DMA,
        ],
    )
    def body(table_hbm, idx_hbm, out_hbm, idx_v, val_v, sem):
        c = lax.axis_index("c")
        s = lax.axis_index("s")
        wid = s * _NC + c
        for k in range(nch):
            base = wid * epw + k * _CHUNK
            pltpu.sync_copy(idx_hbm.at[pl.ds(base, _CHUNK)], idx_v)
            pltpu.async_copy(table_hbm.at[idx_v], val_v, sem).wait()
            pltpu.sync_copy(val_v, out_hbm.at[pl.ds(base, _CHUNK)])

    return body(table, idx)


def _sc_scatter(msg, dst, zeros_2d):
    E, G = msg.shape
    N = zeros_2d.shape[0]
    epw = E // _NW
    nch = epw // _CHUNK
    rows_per_sub = N // _NS

    @functools.partial(
        pl.kernel,
        out_type=jax.ShapeDtypeStruct((_NC, N, G), jnp.float32),
        mesh=_sc_mesh(),
        compiler_params=_SC_PARAMS,
        scratch_types=[
            pltpu.VMEM((_CHUNK,), jnp.int32),
            pltpu.VMEM((_CHUNK, G), jnp.float32),
            pltpu.VMEM_SHARED((N, G), jnp.float32),
        ],
    )
    def body(msg_hbm, dst_hbm, zeros_hbm, out_hbm, idx_v, rows_v, acc_s):
        c = lax.axis_index("c")
        s = lax.axis_index("s")
        wid = s * _NC + c
        r0 = s * rows_per_sub
        pltpu.sync_copy(
            zeros_hbm.at[pl.ds(r0, rows_per_sub)], acc_s.at[pl.ds(r0, rows_per_sub)]
        )
        plsc.subcore_barrier()
        for k in range(nch):
            base = wid * epw + k * _CHUNK
            pltpu.sync_copy(dst_hbm.at[pl.ds(base, _CHUNK)], idx_v)
            pltpu.sync_copy(msg_hbm.at[pl.ds(base, _CHUNK)], rows_v)
            pltpu.sync_copy(rows_v, acc_s.at[idx_v], add=True)
        plsc.subcore_barrier()
        pltpu.sync_copy(
            acc_s.at[pl.ds(r0, rows_per_sub)], out_hbm.at[c, pl.ds(r0, rows_per_sub)]
        )

    return body(msg, dst, zeros_2d)


def _dot(a, b):
    return jnp.dot(a, b, preferred_element_type=jnp.float32)


def _tc_pre(x, W0T, b0row):
    N = x.shape[0]
    G = W0T.shape[1]

    def body(x_ref, w_ref, b_ref, o_ref):
        o_ref[...] = jax.nn.relu(_dot(x_ref[...], w_ref[...]) + b_ref[...])

    return pl.pallas_call(
        body, out_shape=jax.ShapeDtypeStruct((N, G), jnp.float32)
    )(x, W0T, b0row)


def _tc_deginv(deg_p):
    N = deg_p.shape[1]

    def body(d_ref, o_ref):
        d = d_ref[0:1, :] + d_ref[1:2, :]
        o_ref[...] = 1.0 / jnp.maximum(d, 1.0)

    return pl.pallas_call(
        body, out_shape=jax.ShapeDtypeStruct((1, N), jnp.float32)
    )(deg_p)


def _tc_msg(xs, ea, degi, We1T, be1row, We2T, be2row, R, S):
    E, G = xs.shape
    EF = ea.shape[1]
    EH = We1T.shape[1]
    GG = We2T.shape[1]
    tile = _MSG_TILE
    grid = E // tile

    def body(xs_ref, ea_ref, di_ref, w1_ref, b1_ref, w2_ref, b2_ref, r_ref, s_ref, o_ref):
        eh = jax.nn.relu(_dot(ea_ref[...], w1_ref[...]) + b1_ref[...])
        wf = _dot(eh, w2_ref[...]) + b2_ref[...]
        rep = _dot(xs_ref[...], r_ref[...])
        o_ref[...] = _dot(rep * wf, s_ref[...]) * di_ref[...]

    full = lambda shape: pl.BlockSpec(shape, lambda i: (0,) * len(shape))
    return pl.pallas_call(
        body,
        grid=(grid,),
        in_specs=[
            pl.BlockSpec((tile, G), lambda i: (i, 0)),
            pl.BlockSpec((tile, EF), lambda i: (i, 0)),
            pl.BlockSpec((tile, 1), lambda i: (i, 0)),
            full((EF, EH)),
            full((1, EH)),
            full((EH, GG)),
            full((1, GG)),
            full((G, GG)),
            full((GG, G)),
        ],
        out_specs=pl.BlockSpec((tile, G), lambda i: (i, 0)),
        out_shape=jax.ShapeDtypeStruct((E, G), jnp.float32),
    )(xs, ea, degi, We1T, be1row, We2T, be2row, R, S)


def _tc_update(h, p0, p1, WrootM, bconvrow, wih, whh, brz, bn):
    N, G = h.shape

    def body(h_ref, p0_ref, p1_ref, wroot_ref, bc_ref,
             wr_i, wz_i, wn_i, wr_h, wz_h, wn_h, br_ref, bz_ref,
             bn_i_ref, bn_h_ref, o_ref):
        hcur = h_ref[...]
        m = jax.nn.relu(_dot(hcur, wroot_ref[...]) + p0_ref[...] + p1_ref[...] + bc_ref[...])
        r = jax.nn.sigmoid(_dot(m, wr_i[...]) + _dot(hcur, wr_h[...]) + br_ref[...])
        z = jax.nn.sigmoid(_dot(m, wz_i[...]) + _dot(hcur, wz_h[...]) + bz_ref[...])
        n = jnp.tanh(_dot(m, wn_i[...]) + bn_i_ref[...]
                     + r * (_dot(hcur, wn_h[...]) + bn_h_ref[...]))
        o_ref[...] = (1.0 - z) * n + z * hcur

    return pl.pallas_call(
        body, out_shape=jax.ShapeDtypeStruct((N, G), jnp.float32)
    )(h, p0, p1, WrootM, bconvrow, *wih, *whh, *brz, *bn)


def _tc_s2s(out, batch_col, batch_row, num_graphs, wih, whh, bg,
            Wf1T, bf1row, Wf2T, bf2row):
    N, G = out.shape
    GR = whh[0].shape[0]

    def body(o_ref, bc_ref, br_ref,
             wi_i, wi_f, wi_g, wi_o, wh_i, wh_f, wh_g, wh_o,
             b_i, b_f, b_g, b_o, wf1_ref, bf1_ref, wf2_ref, bf2_ref, y_ref):
        o = o_ref[...]
        bcol = bc_ref[...]
        brow = br_ref[...]
        onehot_b = bcol == lax.broadcasted_iota(jnp.int32, (N, num_graphs), 1)
        onehot_f = onehot_b.astype(jnp.float32)
        onehotT_f = (
            lax.broadcasted_iota(jnp.int32, (num_graphs, N), 0) == brow
        ).astype(jnp.float32)

        q_star = jnp.zeros((num_graphs, 2 * GR), jnp.float32)
        hs = jnp.zeros((num_graphs, GR), jnp.float32)
        cs = jnp.zeros((num_graphs, GR), jnp.float32)
        for _ in range(3):
            ig = jax.nn.sigmoid(_dot(q_star, wi_i[...]) + _dot(hs, wh_i[...]) + b_i[...])
            fg = jax.nn.sigmoid(_dot(q_star, wi_f[...]) + _dot(hs, wh_f[...]) + b_f[...])
            gg = jnp.tanh(_dot(q_star, wi_g[...]) + _dot(hs, wh_g[...]) + b_g[...])
            og = jax.nn.sigmoid(_dot(q_star, wi_o[...]) + _dot(hs, wh_o[...]) + b_o[...])
            cs = fg * cs + ig * gg
            hs = og * jnp.tanh(cs)
            q = hs
            qb = _dot(onehot_f, q)
            e_col = jnp.sum(o * qb, axis=1, keepdims=True)
            masked = jnp.where(onehot_b, e_col, -jnp.inf)
            emax_row = jnp.max(masked, axis=0, keepdims=True)
            emax_row = jnp.where(jnp.isfinite(emax_row), emax_row, 0.0)
            emax_b = jnp.max(
                jnp.where(onehot_b, emax_row, -jnp.inf), axis=1, keepdims=True
            )
            ex = jnp.exp(e_col - emax_b)
            denom_row = jnp.sum(onehot_f * ex, axis=0, keepdims=True)
            denom_b = jnp.sum(onehot_f * denom_row, axis=1, keepdims=True)
            a = ex / jnp.maximum(denom_b, 1e-16)
            rvec = _dot(onehotT_f, a * o)
            q_star = jnp.concatenate([q, rvec], axis=1)
        y = _dot(jax.nn.relu(_dot(q_star, wf1_ref[...]) + bf1_ref[...]), wf2_ref[...])
        y_ref[...] = y + bf2_ref[...]

    return pl.pallas_call(
        body, out_shape=jax.ShapeDtypeStruct((num_graphs, 1), jnp.float32)
    )(out, batch_col, batch_row, *wih, *whh, *bg, Wf1T, bf1row, Wf2T, bf2row)


def kernel(x, edge_index, edge_attr, batch, W0, b0, We1, be1, We2, be2, Wroot,
           bconv, gru_Wih, gru_Whh, gru_bih, gru_bhh, lstm_Wih, lstm_Whh,
           lstm_bih, lstm_bhh, Wf1, bf1, Wf2, bf2):
    N, F = x.shape
    E = edge_attr.shape[0]
    G = W0.shape[0]
    num_graphs = 64
    src = edge_index[0]
    dst = edge_index[1]

    zeros_n = jnp.zeros((N,), jnp.float32)
    zeros_2d = jnp.zeros((N, G), jnp.float32)
    ones_c = jnp.ones((_CHUNK,), jnp.float32)

    eye = jnp.eye(G, dtype=jnp.float32)
    R = jnp.kron(eye, jnp.ones((1, G), jnp.float32))
    S = jnp.kron(jnp.ones((G, 1), jnp.float32), eye)

    out0 = _tc_pre(x, W0.T, b0[None, :])

    deg_p = _sc_deg(dst, ones_c, zeros_n)
    dinv_row = _tc_deginv(deg_p)
    degi = _sc_gather_scalar(dinv_row.reshape(N), dst).reshape(E, 1)

    wih = (gru_Wih[0:G].T, gru_Wih[G:2 * G].T, gru_Wih[2 * G:].T)
    whh = (gru_Whh[0:G].T, gru_Whh[G:2 * G].T, gru_Whh[2 * G:].T)
    brz = ((gru_bih[0:G] + gru_bhh[0:G])[None, :],
           (gru_bih[G:2 * G] + gru_bhh[G:2 * G])[None, :])
    bn = (gru_bih[2 * G:][None, :], gru_bhh[2 * G:][None, :])

    h = out0
    for _ in range(3):
        xs = _sc_gather_rows(h, src)
        msg = _tc_msg(xs, edge_attr, degi, We1.T, be1[None, :], We2.T,
                      be2[None, :], R, S)
        parts = _sc_scatter(msg, dst, zeros_2d)
        h = _tc_update(h, parts[0], parts[1], Wroot, bconv[None, :],
                       wih, whh, brz, bn)

    GR = gru_Whh.shape[1]
    lwih = (lstm_Wih[0:GR].T, lstm_Wih[GR:2 * GR].T,
            lstm_Wih[2 * GR:3 * GR].T, lstm_Wih[3 * GR:].T)
    lwhh = (lstm_Whh[0:GR].T, lstm_Whh[GR:2 * GR].T,
            lstm_Whh[2 * GR:3 * GR].T, lstm_Whh[3 * GR:].T)
    lb = ((lstm_bih[0:GR] + lstm_bhh[0:GR])[None, :],
          (lstm_bih[GR:2 * GR] + lstm_bhh[GR:2 * GR])[None, :],
          (lstm_bih[2 * GR:3 * GR] + lstm_bhh[2 * GR:3 * GR])[None, :],
          (lstm_bih[3 * GR:] + lstm_bhh[3 * GR:])[None, :])

    y = _tc_s2s(h, batch.reshape(N, 1), batch.reshape(1, N), num_graphs,
                lwih, lwhh, lb, Wf1.T, bf1[None, :], Wf2.T, bf2[None, :])
    return y
